# baseline (speedup 1.0000x reference)
import jax
import jax.numpy as jnp
from jax.experimental import pallas as pl
from jax.experimental.pallas import tpu as pltpu

_D = 1024
_B = 4
_LN_EPS = 1e-5
_TS = 256


def _ln_kernel(x_ref, pe_ref, g_ref, b_ref, o_ref, pe3_ref, sems):
    copies = [
        pltpu.make_async_copy(pe_ref, pe3_ref.at[:, i, :], sems.at[i])
        for i in range(_B)
    ]
    for c in copies:
        c.start()
    for c in copies:
        c.wait()
    g = g_ref[...]
    b = b_ref[...]
    h = x_ref[...] + pe3_ref[...]
    mean = jnp.mean(h, axis=-1, keepdims=True)
    hc = h - mean
    var = jnp.mean(hc * hc, axis=-1, keepdims=True)
    o_ref[...] = hc * jax.lax.rsqrt(var + _LN_EPS) * g[None] + b[None]


def kernel(x, pos_table, ln_gamma, ln_beta):
    S, B, D = x.shape
    g2 = ln_gamma.reshape(1, D)
    b2 = ln_beta.reshape(1, D)
    out = pl.pallas_call(
        _ln_kernel,
        grid=(S // _TS,),
        in_specs=[
            pl.BlockSpec((_TS, B, D), lambda s: (s, 0, 0)),
            pl.BlockSpec((_TS, D), lambda s: (s, 0)),
            pl.BlockSpec((1, D), lambda s: (0, 0)),
            pl.BlockSpec((1, D), lambda s: (0, 0)),
        ],
        out_specs=pl.BlockSpec((_TS, B, D), lambda s: (s, 0, 0)),
        out_shape=jax.ShapeDtypeStruct((S, B, D), x.dtype),
        scratch_shapes=[
            pltpu.VMEM((_TS, B, D), jnp.float32),
            pltpu.SemaphoreType.DMA((_B,)),
        ],
        compiler_params=pltpu.CompilerParams(
            dimension_semantics=("arbitrary",)),
    )(x, pos_table, g2, b2)
    return out


# speedup vs baseline: 1.5663x; 1.5663x over previous
import jax
import jax.numpy as jnp
from jax.experimental import pallas as pl
from jax.experimental.pallas import tpu as pltpu

_D = 1024
_B = 4
_LN_EPS = 1e-5
_TS = 512


def _ln_kernel(x_ref, pe_ref, g_ref, b_ref, o_ref, pe3_ref, sem):
    copy = pltpu.make_async_copy(pe_ref, pe3_ref.at[:, 0, :], sem)
    copy.start()
    copy.wait()
    g = g_ref[...]
    b = b_ref[...]
    h = x_ref[...] + pltpu.repeat(pe3_ref[...], _B, axis=1)
    mean = jnp.mean(h, axis=-1, keepdims=True)
    hc = h - mean
    var = jnp.mean(hc * hc, axis=-1, keepdims=True)
    o_ref[...] = hc * jax.lax.rsqrt(var + _LN_EPS) * g[None] + b[None]


def kernel(x, pos_table, ln_gamma, ln_beta):
    S, B, D = x.shape
    g2 = ln_gamma.reshape(1, D)
    b2 = ln_beta.reshape(1, D)
    out = pl.pallas_call(
        _ln_kernel,
        grid=(S // _TS,),
        in_specs=[
            pl.BlockSpec((_TS, B, D), lambda s: (s, 0, 0)),
            pl.BlockSpec((_TS, D), lambda s: (s, 0)),
            pl.BlockSpec((1, D), lambda s: (0, 0)),
            pl.BlockSpec((1, D), lambda s: (0, 0)),
        ],
        out_specs=pl.BlockSpec((_TS, B, D), lambda s: (s, 0, 0)),
        out_shape=jax.ShapeDtypeStruct((S, B, D), x.dtype),
        scratch_shapes=[
            pltpu.VMEM((_TS, 1, D), jnp.float32),
            pltpu.SemaphoreType.DMA,
        ],
        compiler_params=pltpu.CompilerParams(
            dimension_semantics=("arbitrary",)),
    )(x, pos_table, g2, b2)
    return out


# speedup vs baseline: 1.5792x; 1.0082x over previous
import jax
import jax.numpy as jnp
from jax.experimental import pallas as pl
from jax.experimental.pallas import tpu as pltpu

_D = 1024
_B = 4
_LN_EPS = 1e-5
_TS = 512
_CH = 64


def _ln_kernel(x_ref, pe_ref, g_ref, b_ref, o_ref):
    g = g_ref[...]
    b = b_ref[...]
    for c in range(_TS // _CH):
        sl = pl.ds(c * _CH, _CH)
        pe = pe_ref[sl, :]
        h = x_ref[sl, :, :] + pe[:, None, :]
        mean = jnp.mean(h, axis=-1, keepdims=True)
        hc = h - mean
        var = jnp.mean(hc * hc, axis=-1, keepdims=True)
        o_ref[sl, :, :] = hc * jax.lax.rsqrt(var + _LN_EPS) * g[None] + b[None]


def kernel(x, pos_table, ln_gamma, ln_beta):
    S, B, D = x.shape
    g2 = ln_gamma.reshape(1, D)
    b2 = ln_beta.reshape(1, D)
    out = pl.pallas_call(
        _ln_kernel,
        grid=(S // _TS,),
        in_specs=[
            pl.BlockSpec((_TS, B, D), lambda s: (s, 0, 0)),
            pl.BlockSpec((_TS, D), lambda s: (s, 0)),
            pl.BlockSpec((1, D), lambda s: (0, 0)),
            pl.BlockSpec((1, D), lambda s: (0, 0)),
        ],
        out_specs=pl.BlockSpec((_TS, B, D), lambda s: (s, 0, 0)),
        out_shape=jax.ShapeDtypeStruct((S, B, D), x.dtype),
        compiler_params=pltpu.CompilerParams(
            dimension_semantics=("arbitrary",)),
    )(x, pos_table, g2, b2)
    return out
